# Initial kernel scaffold; baseline (speedup 1.0000x reference)
import jax
import jax.numpy as jnp
from jax.experimental import pallas as pl


def kernel(x):
    raise NotImplementedError("write your pallas kernel here")


# speedup vs baseline: 1.3214x; 1.3214x over previous
import functools
import numpy as np
import jax
import jax.numpy as jnp
from jax.experimental import pallas as pl
from jax.experimental.pallas import tpu as pltpu

_N = 64
_Q_FRACS = np.asarray([0.1, 0.2, 0.3, 0.4, 0.5, 0.6, 0.7, 0.8, 0.9], np.float32)
_IDX = np.round(_Q_FRACS * 100.0 / 100.0 * (_N - 1)).astype(np.int32)
_NQ = _IDX.shape[0]

_SEL = np.zeros((128, 2 * _NQ), np.float32)
for _q, _i in enumerate(_IDX):
    _SEL[_i, _q] = 1.0
    _SEL[64 + _i, _NQ + _q] = 1.0


def _bitonic_body(sel_ref, x_ref, o_ref):
    x = x_ref[...]
    rows = x.shape[0]
    lane = jax.lax.broadcasted_iota(jnp.int32, (rows, 128), 1)
    for k_log in range(1, 7):
        k = 1 << k_log
        for j_log in range(k_log - 1, -1, -1):
            j = 1 << j_log
            bit_j = (lane & j) != 0
            partner = jnp.where(bit_j, pltpu.roll(x, j, 1), pltpu.roll(x, 128 - j, 1))
            if k == _N:
                take_min = jnp.logical_not(bit_j)
            else:
                take_min = jnp.logical_not(bit_j) == ((lane & k) == 0)
            x = jnp.where(take_min, jnp.minimum(x, partner), jnp.maximum(x, partner))
    o_ref[...] = jnp.dot(x, sel_ref[...], preferred_element_type=jnp.float32,
                         precision=jax.lax.Precision.HIGHEST)


def kernel(x):
    b, t, n = x.shape
    rows = b * t // 2
    xv = x.reshape(rows, 2 * n)
    blk = 2048
    grid = rows // blk
    out = pl.pallas_call(
        _bitonic_body,
        grid=(grid,),
        in_specs=[
            pl.BlockSpec((128, 2 * _NQ), lambda i: (0, 0)),
            pl.BlockSpec((blk, 2 * n), lambda i: (i, 0)),
        ],
        out_specs=pl.BlockSpec((blk, 2 * _NQ), lambda i: (i, 0)),
        out_shape=jax.ShapeDtypeStruct((rows, 2 * _NQ), jnp.float32),
    )(jnp.asarray(_SEL), xv)
    return out.reshape(b, t, _NQ)


# speedup vs baseline: 3.8411x; 2.9069x over previous
import numpy as np
import jax
import jax.numpy as jnp
from jax.experimental import pallas as pl
from jax.experimental.pallas import tpu as pltpu

_N = 64
_Q_FRACS = np.asarray([0.1, 0.2, 0.3, 0.4, 0.5, 0.6, 0.7, 0.8, 0.9], np.float32)
_IDX = np.round(_Q_FRACS * (_N - 1)).astype(np.int32)
_NQ = _IDX.shape[0]


def _bitrev6(v):
    r = 0
    for b in range(6):
        r |= ((v >> b) & 1) << (5 - b)
    return r


_SEL = np.zeros((_NQ, _N), np.float32)
for _q, _r in enumerate(_IDX):
    _SEL[_q, _bitrev6(int(_r))] = 1.0

_SUB = 256
_NSUB = 4
_BLK = _SUB * _NSUB


def _sort_subtile(v, ri):
    for big_k in range(1, 7):
        kb = 1 << (5 - big_k) if big_k < 6 else 0
        for big_j in range(big_k - 1, -1, -1):
            sd = 1 << (5 - big_j)
            if sd >= 8:
                p = sd // 8
                g2 = 8 // (2 * p)
                v5 = v.reshape(g2, 2, p, 8, _SUB)
                lo, hi = v5[:, 0], v5[:, 1]
                mn, mx = jnp.minimum(lo, hi), jnp.maximum(lo, hi)
                if kb:
                    m = ((ri & kb) == 0).reshape(g2, 2, p, 8, _SUB)[:, 0]
                    lo2 = jnp.where(m, mn, mx)
                    hi2 = jnp.where(m, mx, mn)
                else:
                    lo2, hi2 = mn, mx
                v = jnp.concatenate([lo2[:, None], hi2[:, None]], axis=1)
                v = v.reshape(_N, _SUB)
            else:
                up = pltpu.roll(v, sd, 0)
                dn = pltpu.roll(v, _N - sd, 0)
                bit = (ri & sd) != 0
                partner = jnp.where(bit, up, dn)
                tm = jnp.logical_not(bit)
                if kb:
                    tm = tm == ((ri & kb) == 0)
                v = jnp.where(tm, jnp.minimum(v, partner), jnp.maximum(v, partner))
    return v


def _body(sel_ref, x_ref, o_ref):
    ri = jax.lax.broadcasted_iota(jnp.int32, (_N, _SUB), 0)
    sel = sel_ref[...]
    for s in range(_NSUB):
        v = jnp.transpose(x_ref[pl.ds(s * _SUB, _SUB), :], (1, 0))
        v = _sort_subtile(v, ri)
        o_ref[:, pl.ds(s * _SUB, _SUB)] = jnp.dot(
            sel, v, preferred_element_type=jnp.float32,
            precision=jax.lax.Precision.HIGHEST)


def kernel(x):
    b, t, n = x.shape
    rows = b * t
    xv = x.reshape(rows, n)
    grid = rows // _BLK
    out = pl.pallas_call(
        _body,
        grid=(grid,),
        in_specs=[
            pl.BlockSpec((_NQ, _N), lambda i: (0, 0)),
            pl.BlockSpec((_BLK, _N), lambda i: (i, 0)),
        ],
        out_specs=pl.BlockSpec((_NQ, _BLK), lambda i: (0, i)),
        out_shape=jax.ShapeDtypeStruct((_NQ, rows), jnp.float32),
    )(jnp.asarray(_SEL), xv)
    return jnp.transpose(out, (1, 0)).reshape(b, t, _NQ)
